# Initial kernel scaffold; baseline (speedup 1.0000x reference)
import jax
import jax.numpy as jnp
from jax.experimental import pallas as pl


def kernel(database, query, input_features, filter_weights, nn_index, nn_count, nn_dist):
    raise NotImplementedError("write your pallas kernel here")


# speedup vs baseline: 35.5095x; 35.5095x over previous
import functools
import math

import jax
import jax.numpy as jnp
from jax import lax
from jax.experimental import pallas as pl
from jax.experimental.pallas import tpu as pltpu
from jax.experimental.pallas import tpu_sc as plsc

N_AZ, N_EL, N_RAD = 4, 2, 2
RADIUS = 0.05
AZ_SCALE = N_AZ / (2.0 * math.pi)
EL_SCALE = N_EL / math.pi
NBINS = N_AZ * N_EL * N_RAD

_CH = 128


def _sc_gather(dbp, feat, idx):
    R = idx.shape[0]
    info = plsc.get_sparse_core_info()
    nc = info.num_cores
    nw = nc * info.num_subcores
    per_w = R // nw
    n_ch = per_w // _CH

    mesh = plsc.VectorSubcoreMesh(core_axis_name="c", subcore_axis_name="s")

    @functools.partial(
        pl.kernel,
        mesh=mesh,
        out_type=(
            jax.ShapeDtypeStruct((R, 8), jnp.float32),
            jax.ShapeDtypeStruct((R, 16), jnp.float32),
        ),
        scratch_types=[
            pltpu.VMEM((per_w,), jnp.int32),
            pltpu.VMEM((_CH, 8), jnp.float32),
            pltpu.VMEM((_CH, 16), jnp.float32),
            pltpu.SemaphoreType.DMA,
            pltpu.SemaphoreType.DMA,
        ],
        compiler_params=pltpu.CompilerParams(use_tc_tiling_on_sc=False),
    )
    def k(dbp_hbm, feat_hbm, idx_hbm, outp_hbm, outf_hbm,
          idx_v, pos_v, fea_v, sem1, sem2):
        wid = lax.axis_index("s") * nc + lax.axis_index("c")
        base = wid * per_w
        pltpu.sync_copy(idx_hbm.at[pl.ds(base, per_w)], idx_v)

        def body(g, carry):
            off = g * _CH
            cp1 = pltpu.async_copy(
                dbp_hbm.at[idx_v.at[pl.ds(off, _CH)]], pos_v, sem1)
            cp2 = pltpu.async_copy(
                feat_hbm.at[idx_v.at[pl.ds(off, _CH)]], fea_v, sem2)
            cp1.wait()
            cp2.wait()
            pltpu.sync_copy(pos_v, outp_hbm.at[pl.ds(base + off, _CH)])
            pltpu.sync_copy(fea_v, outf_hbm.at[pl.ds(base + off, _CH)])
            return carry

        lax.fori_loop(0, n_ch, body, 0)

    return k(dbp, feat, idx)


def _tc_body(pos_ref, fea_ref, dist_ref, qry_ref, bw_ref, out_ref):
    f32 = jnp.float32
    Q = pos_ref.shape[0]
    pos = pos_ref[...]
    i0 = lax.broadcasted_iota(jnp.int32, (128, 16), 0)
    i1 = lax.broadcasted_iota(jnp.int32, (128, 16), 1)
    sx = (i0 == i1 * 8).astype(f32)
    sy = (i0 == i1 * 8 + 1).astype(f32)
    sz = (i0 == i1 * 8 + 2).astype(f32)
    x = jnp.dot(pos, sx, preferred_element_type=f32, precision=lax.Precision.HIGHEST)
    y = jnp.dot(pos, sy, preferred_element_type=f32, precision=lax.Precision.HIGHEST)
    z = jnp.dot(pos, sz, preferred_element_type=f32, precision=lax.Precision.HIGHEST)
    q = qry_ref[...]
    dx = x - q[:, 0:1]
    dy = y - q[:, 1:2]
    dz = z - q[:, 2:3]
    d = dist_ref[...]
    az = jnp.arctan2(dy, dx) + math.pi
    cz = jnp.clip(dz / (d + 1e-8), -1.0, 1.0)
    el = jnp.arctan2(jnp.sqrt(jnp.maximum(1.0 - cz * cz, 0.0)), cz)
    ab = az * AZ_SCALE
    eb = el * EL_SCALE
    rb = jnp.clip(d / RADIUS, 0.0, N_RAD - 1e-6)
    af = jnp.floor(ab)
    ef = jnp.floor(eb)
    rf = jnp.floor(rb)
    a_fr = ab - af
    e_fr = eb - ef
    r_fr = rb - rf
    a_in = 1.0 - a_fr
    e_in = 1.0 - e_fr
    r_in = 1.0 - r_fr
    ai = af.astype(jnp.int32)
    ei = ef.astype(jnp.int32)
    ri = rf.astype(jnp.int32)
    e0 = jnp.clip(ei, 0, N_EL - 1)
    e1 = jnp.clip(ei + 1, 0, N_EL - 1)
    r0 = jnp.clip(ri, 0, N_RAD - 1)
    r1 = jnp.clip(ri + 1, 0, N_RAD - 1)
    b0 = (ai % N_AZ) * N_EL
    b1 = ((ai + 1) % N_AZ) * N_EL
    coeffs = (a_in * e_in * r_in, a_fr * e_in * r_in,
              a_in * e_fr * r_in, a_fr * e_fr * r_in,
              a_in * e_in * r_fr, a_fr * e_in * r_fr,
              a_in * e_fr * r_fr, a_fr * e_fr * r_fr)
    bins = ((b0 + e0) * N_RAD + r0, (b1 + e0) * N_RAD + r0,
            (b0 + e1) * N_RAD + r0, (b1 + e1) * N_RAD + r0,
            (b0 + e0) * N_RAD + r1, (b1 + e0) * N_RAD + r1,
            (b0 + e1) * N_RAD + r1, (b1 + e1) * N_RAD + r1)
    j0 = lax.broadcasted_iota(jnp.int32, (16, 256), 0)
    j1 = lax.broadcasted_iota(jnp.int32, (16, 256), 1)
    ex = ((j1 // 16) == j0).astype(f32)
    band = (lax.broadcasted_iota(jnp.int32, (Q, 256), 1) % 16).astype(f32)
    wb = jnp.zeros((Q, 256), f32)
    for cj, bj in zip(coeffs, bins):
        cj_rep = jnp.dot(cj, ex, preferred_element_type=f32, precision=lax.Precision.HIGHEST)
        bj_rep = jnp.dot(bj.astype(f32), ex, preferred_element_type=f32, precision=lax.Precision.HIGHEST)
        wb = wb + cj_rep * (bj_rep == band).astype(f32)
    s = jnp.dot(wb, bw_ref[...], preferred_element_type=f32, precision=lax.Precision.HIGHEST)
    prod = fea_ref[...] * s
    k0 = lax.broadcasted_iota(jnp.int32, (256, 16), 0)
    k1 = lax.broadcasted_iota(jnp.int32, (256, 16), 1)
    rm = ((k0 % 16) == k1).astype(f32)
    out_ref[...] = jnp.dot(prod, rm, preferred_element_type=f32, precision=lax.Precision.HIGHEST)


def _tc_combine(posb, featb, distb, queryb, bw):
    BM = distb.shape[0]
    Q = 1024
    return pl.pallas_call(
        _tc_body,
        grid=(BM // Q,),
        in_specs=[
            pl.BlockSpec((Q, 128), lambda i: (i, 0)),
            pl.BlockSpec((Q, 256), lambda i: (i, 0)),
            pl.BlockSpec((Q, 16), lambda i: (i, 0)),
            pl.BlockSpec((Q, 3), lambda i: (i, 0)),
            pl.BlockSpec((256, 256), lambda i: (0, 0)),
        ],
        out_specs=pl.BlockSpec((Q, 16), lambda i: (i, 0)),
        out_shape=jax.ShapeDtypeStruct((BM, 16), jnp.float32),
    )(posb, featb, distb, queryb, bw)


def kernel(database, query, input_features, filter_weights,
           nn_index, nn_count, nn_dist):
    B, N, _ = database.shape
    _, M, K = nn_index.shape
    C = input_features.shape[-1]
    BM = B * M
    R = BM * K
    dbp = jnp.pad(database, ((0, 0), (0, 0), (0, 5))).reshape(B * N, 8)
    feat_t = input_features.reshape(B * N, C)
    idx = (nn_index
           + (jnp.arange(B, dtype=jnp.int32) * N)[:, None, None]).reshape(R)
    posg, featg = _sc_gather(dbp, feat_t, idx)
    posb = posg.reshape(BM, K * 8)
    featb = featg.reshape(BM, K * C)
    distb = nn_dist.reshape(BM, K)
    queryb = query.reshape(BM, 3)
    w = filter_weights.reshape(NBINS, C)
    bw = jnp.kron(jnp.eye(K, dtype=jnp.float32), w)
    out = _tc_combine(posb, featb, distb, queryb, bw)
    return out.reshape(B, M, C)


# speedup vs baseline: 38.1287x; 1.0738x over previous
import functools
import math

import jax
import jax.numpy as jnp
from jax import lax
from jax.experimental import pallas as pl
from jax.experimental.pallas import tpu as pltpu
from jax.experimental.pallas import tpu_sc as plsc

N_AZ, N_EL, N_RAD = 4, 2, 2
RADIUS = 0.05
AZ_SCALE = N_AZ / (2.0 * math.pi)
EL_SCALE = N_EL / math.pi
NBINS = N_AZ * N_EL * N_RAD

_CH = 128
_NB = 4


def _sc_gather(dbp, feat, idx):
    R = idx.shape[0]
    info = plsc.get_sparse_core_info()
    nc = info.num_cores
    nw = nc * info.num_subcores
    per_w = R // nw
    n_ch = per_w // _CH

    mesh = plsc.VectorSubcoreMesh(core_axis_name="c", subcore_axis_name="s")

    @functools.partial(
        pl.kernel,
        mesh=mesh,
        out_type=(
            jax.ShapeDtypeStruct((R, 8), jnp.float32),
            jax.ShapeDtypeStruct((R, 16), jnp.float32),
        ),
        scratch_types=[
            pltpu.VMEM((per_w,), jnp.int32),
            pltpu.VMEM((_NB, _CH, 8), jnp.float32),
            pltpu.VMEM((_NB, _CH, 16), jnp.float32),
            pltpu.SemaphoreType.DMA,
            pltpu.SemaphoreType.DMA,
        ],
        compiler_params=pltpu.CompilerParams(use_tc_tiling_on_sc=False),
    )
    def k(dbp_hbm, feat_hbm, idx_hbm, outp_hbm, outf_hbm,
          idx_v, pos_v, fea_v, sem1, sem2):
        wid = lax.axis_index("s") * nc + lax.axis_index("c")
        base = wid * per_w
        pltpu.sync_copy(idx_hbm.at[pl.ds(base, per_w)], idx_v)

        def body(t, carry):
            cps = []
            for b in range(_NB):
                off = (t * _NB + b) * _CH
                cps.append(pltpu.async_copy(
                    dbp_hbm.at[idx_v.at[pl.ds(off, _CH)]], pos_v.at[b], sem1))
                cps.append(pltpu.async_copy(
                    feat_hbm.at[idx_v.at[pl.ds(off, _CH)]], fea_v.at[b], sem2))
            for cp in cps:
                cp.wait()
            for b in range(_NB):
                off = (t * _NB + b) * _CH
                pltpu.sync_copy(pos_v.at[b], outp_hbm.at[pl.ds(base + off, _CH)])
                pltpu.sync_copy(fea_v.at[b], outf_hbm.at[pl.ds(base + off, _CH)])
            return carry

        lax.fori_loop(0, n_ch // _NB, body, 0)

    return k(dbp, feat, idx)


def _tc_body(pos_ref, fea_ref, dist_ref, qry_ref, bw_ref, out_ref):
    f32 = jnp.float32
    Q = pos_ref.shape[0]
    pos = pos_ref[...]
    i0 = lax.broadcasted_iota(jnp.int32, (128, 16), 0)
    i1 = lax.broadcasted_iota(jnp.int32, (128, 16), 1)
    sx = (i0 == i1 * 8).astype(f32)
    sy = (i0 == i1 * 8 + 1).astype(f32)
    sz = (i0 == i1 * 8 + 2).astype(f32)
    x = jnp.dot(pos, sx, preferred_element_type=f32, precision=lax.Precision.HIGHEST)
    y = jnp.dot(pos, sy, preferred_element_type=f32, precision=lax.Precision.HIGHEST)
    z = jnp.dot(pos, sz, preferred_element_type=f32, precision=lax.Precision.HIGHEST)
    q = qry_ref[...]
    dx = x - q[:, 0:1]
    dy = y - q[:, 1:2]
    dz = z - q[:, 2:3]
    d = dist_ref[...]
    az = jnp.arctan2(dy, dx) + math.pi
    cz = jnp.clip(dz / (d + 1e-8), -1.0, 1.0)
    el = jnp.arctan2(jnp.sqrt(jnp.maximum(1.0 - cz * cz, 0.0)), cz)
    ab = az * AZ_SCALE
    eb = el * EL_SCALE
    rb = jnp.clip(d / RADIUS, 0.0, N_RAD - 1e-6)
    af = jnp.floor(ab)
    ef = jnp.floor(eb)
    rf = jnp.floor(rb)
    a_fr = ab - af
    e_fr = eb - ef
    r_fr = rb - rf
    a_in = 1.0 - a_fr
    e_in = 1.0 - e_fr
    r_in = 1.0 - r_fr
    ai = af.astype(jnp.int32)
    ei = ef.astype(jnp.int32)
    ri = rf.astype(jnp.int32)
    e0 = jnp.clip(ei, 0, N_EL - 1)
    e1 = jnp.clip(ei + 1, 0, N_EL - 1)
    r0 = jnp.clip(ri, 0, N_RAD - 1)
    r1 = jnp.clip(ri + 1, 0, N_RAD - 1)
    b0 = (ai % N_AZ) * N_EL
    b1 = ((ai + 1) % N_AZ) * N_EL
    coeffs = (a_in * e_in * r_in, a_fr * e_in * r_in,
              a_in * e_fr * r_in, a_fr * e_fr * r_in,
              a_in * e_in * r_fr, a_fr * e_in * r_fr,
              a_in * e_fr * r_fr, a_fr * e_fr * r_fr)
    bins = ((b0 + e0) * N_RAD + r0, (b1 + e0) * N_RAD + r0,
            (b0 + e1) * N_RAD + r0, (b1 + e1) * N_RAD + r0,
            (b0 + e0) * N_RAD + r1, (b1 + e0) * N_RAD + r1,
            (b0 + e1) * N_RAD + r1, (b1 + e1) * N_RAD + r1)
    j0 = lax.broadcasted_iota(jnp.int32, (16, 256), 0)
    j1 = lax.broadcasted_iota(jnp.int32, (16, 256), 1)
    ex = ((j1 // 16) == j0).astype(f32)
    band = (lax.broadcasted_iota(jnp.int32, (Q, 256), 1) % 16).astype(f32)
    wb = jnp.zeros((Q, 256), f32)
    for cj, bj in zip(coeffs, bins):
        cj_rep = jnp.dot(cj, ex, preferred_element_type=f32, precision=lax.Precision.HIGHEST)
        bj_rep = jnp.dot(bj.astype(f32), ex, preferred_element_type=f32, precision=lax.Precision.HIGHEST)
        wb = wb + cj_rep * (bj_rep == band).astype(f32)
    s = jnp.dot(wb, bw_ref[...], preferred_element_type=f32, precision=lax.Precision.HIGHEST)
    prod = fea_ref[...] * s
    k0 = lax.broadcasted_iota(jnp.int32, (256, 16), 0)
    k1 = lax.broadcasted_iota(jnp.int32, (256, 16), 1)
    rm = ((k0 % 16) == k1).astype(f32)
    out_ref[...] = jnp.dot(prod, rm, preferred_element_type=f32, precision=lax.Precision.HIGHEST)


def _tc_combine(posb, featb, distb, queryb, bw):
    BM = distb.shape[0]
    Q = 1024
    return pl.pallas_call(
        _tc_body,
        grid=(BM // Q,),
        in_specs=[
            pl.BlockSpec((Q, 128), lambda i: (i, 0)),
            pl.BlockSpec((Q, 256), lambda i: (i, 0)),
            pl.BlockSpec((Q, 16), lambda i: (i, 0)),
            pl.BlockSpec((Q, 3), lambda i: (i, 0)),
            pl.BlockSpec((256, 256), lambda i: (0, 0)),
        ],
        out_specs=pl.BlockSpec((Q, 16), lambda i: (i, 0)),
        out_shape=jax.ShapeDtypeStruct((BM, 16), jnp.float32),
    )(posb, featb, distb, queryb, bw)


def kernel(database, query, input_features, filter_weights,
           nn_index, nn_count, nn_dist):
    B, N, _ = database.shape
    _, M, K = nn_index.shape
    C = input_features.shape[-1]
    BM = B * M
    R = BM * K
    dbp = jnp.pad(database, ((0, 0), (0, 0), (0, 5))).reshape(B * N, 8)
    feat_t = input_features.reshape(B * N, C)
    idx = (nn_index
           + (jnp.arange(B, dtype=jnp.int32) * N)[:, None, None]).reshape(R)
    posg, featg = _sc_gather(dbp, feat_t, idx)
    posb = posg.reshape(BM, K * 8)
    featb = featg.reshape(BM, K * C)
    distb = nn_dist.reshape(BM, K)
    queryb = query.reshape(BM, 3)
    w = filter_weights.reshape(NBINS, C)
    bw = jnp.kron(jnp.eye(K, dtype=jnp.float32), w)
    out = _tc_combine(posb, featb, distb, queryb, bw)
    return out.reshape(B, M, C)
